# baseline (speedup 1.0000x reference)
import functools

import jax
import jax.numpy as jnp
from jax import lax
from jax.experimental import pallas as pl
from jax.experimental.pallas import tpu as pltpu
from jax.experimental.pallas import tpu_sc as plsc

E = 8
K = 2
D = 1024
HID = 4096
NEG = -1e9

T_TOK = 256
T_ROW = 256
HBLK = 512
NH = HID // HBLK
N_TOK = 2 * 2048
CAP = K * N_TOK + E * T_ROW
P_TILES = CAP // T_ROW
GCHUNK = 64


def _router_body(x_ref, wg_ref, wn_ref, z_ref,
                 i1_ref, i2_ref, g1_ref, g2_ref, score_ref, fp_ref):
    t = pl.program_id(0)
    nt = pl.num_programs(0)
    x = x_ref[...]
    hx = jnp.dot(x, wg_ref[...], preferred_element_type=jnp.float32)
    v = jnp.dot(x, wn_ref[...], preferred_element_type=jnp.float32)
    softplus = jnp.maximum(v, 0.0) + jnp.log1p(jnp.exp(-jnp.abs(v)))
    hx = hx + z_ref[...] * softplus

    lane = lax.broadcasted_iota(jnp.int32, hx.shape, 1)
    m1 = jnp.max(hx, axis=1, keepdims=True)
    i1 = jnp.min(jnp.where(hx == m1, lane, E), axis=1, keepdims=True)
    hx2 = jnp.where(lane == i1, -jnp.inf, hx)
    m2 = jnp.max(hx2, axis=1, keepdims=True)
    i2 = jnp.min(jnp.where(hx2 == m2, lane, E), axis=1, keepdims=True)

    keep = (lane == i1) | (lane == i2)
    masked = jnp.where(keep, hx, NEG)
    ex = jnp.exp(masked - m1)
    g = ex / jnp.sum(ex, axis=1, keepdims=True)
    g1 = jnp.sum(jnp.where(lane == i1, g, 0.0), axis=1, keepdims=True)
    g2 = jnp.sum(jnp.where(lane == i2, g, 0.0), axis=1, keepdims=True)

    i1_ref[...] = i1
    i2_ref[...] = i2
    g1_ref[...] = g1
    g2_ref[...] = g2

    f_part = jnp.sum((g > 0).astype(jnp.float32), axis=0, keepdims=True)
    p_part = jnp.sum(g, axis=0, keepdims=True)

    @pl.when(t == 0)
    def _():
        fp_ref[...] = jnp.zeros_like(fp_ref)

    fp_ref[0:1, :] += f_part
    fp_ref[1:2, :] += p_part

    @pl.when(t == nt - 1)
    def _():
        f = fp_ref[0:1, :]
        p = fp_ref[1:2, :]
        total = jnp.sum(f * p, keepdims=True) * (E / (K * float(N_TOK) ** 2))
        score_ref[...] = total.reshape(1, 1) - 1.0


def _run_router(xf, w_g, w_n, z):
    nt = N_TOK // T_TOK
    return pl.pallas_call(
        _router_body,
        grid=(nt,),
        in_specs=[
            pl.BlockSpec((T_TOK, D), lambda t: (t, 0)),
            pl.BlockSpec((D, E), lambda t: (0, 0)),
            pl.BlockSpec((D, E), lambda t: (0, 0)),
            pl.BlockSpec((T_TOK, E), lambda t: (t, 0)),
        ],
        out_specs=[
            pl.BlockSpec((T_TOK, 1), lambda t: (t, 0)),
            pl.BlockSpec((T_TOK, 1), lambda t: (t, 0)),
            pl.BlockSpec((T_TOK, 1), lambda t: (t, 0)),
            pl.BlockSpec((T_TOK, 1), lambda t: (t, 0)),
            pl.BlockSpec((1, 1), lambda t: (0, 0)),
        ],
        out_shape=[
            jax.ShapeDtypeStruct((N_TOK, 1), jnp.int32),
            jax.ShapeDtypeStruct((N_TOK, 1), jnp.int32),
            jax.ShapeDtypeStruct((N_TOK, 1), jnp.float32),
            jax.ShapeDtypeStruct((N_TOK, 1), jnp.float32),
            jax.ShapeDtypeStruct((1, 1), jnp.float32),
        ],
        scratch_shapes=[pltpu.VMEM((2, E), jnp.float32)],
    )(xf, w_g, w_n, z)


def _gather_rows(table, idx):
    rows = idx.shape[0]
    width = table.shape[1]
    nw = 32
    b_per_w = rows // nw
    nchunks = b_per_w // GCHUNK
    mesh = plsc.VectorSubcoreMesh(core_axis_name="c", subcore_axis_name="s")

    @functools.partial(
        pl.kernel,
        mesh=mesh,
        out_type=jax.ShapeDtypeStruct((rows, width), jnp.float32),
        scratch_types=[
            pltpu.VMEM((GCHUNK,), jnp.int32),
            pltpu.VMEM((GCHUNK, width), jnp.float32),
            pltpu.SemaphoreType.DMA,
        ],
    )
    def gk(tab_hbm, idx_hbm, out_hbm, idx_v, rows_v, sem):
        wid = lax.axis_index("s") * 2 + lax.axis_index("c")
        base = wid * b_per_w

        def body(c, _):
            o = base + c * GCHUNK
            pltpu.sync_copy(idx_hbm.at[pl.ds(o, GCHUNK)], idx_v)
            pltpu.async_copy(tab_hbm.at[idx_v], rows_v, sem).wait()
            pltpu.sync_copy(rows_v, out_hbm.at[pl.ds(o, GCHUNK)])
            return _

        lax.fori_loop(0, nchunks, body, 0)

    return gk(table, idx)


def _ffn_body(ex_ref, live_ref, xs_ref, w1_ref, b1_ref, w2_ref, b2_ref,
              gate_ref, h_ref):
    p = pl.program_id(0)
    hb = pl.program_id(1)

    @pl.when(live_ref[p] == 1)
    def _():
        x = xs_ref[...]
        h1 = lax.dot_general(x, w1_ref[0], (((1,), (1,)), ((), ())),
                             preferred_element_type=jnp.float32)
        h1 = jnp.maximum(h1 + b1_ref[0, 0], 0.0)
        part = lax.dot_general(h1, w2_ref[0], (((1,), (1,)), ((), ())),
                               preferred_element_type=jnp.float32)

        @pl.when(hb == 0)
        def _():
            h_ref[...] = part

        @pl.when(hb > 0)
        def _():
            h_ref[...] += part

        @pl.when(hb == NH - 1)
        def _():
            h_ref[...] = (h_ref[...] + b2_ref[0]) * gate_ref[...]


def _run_ffn(xs, w1, b1, w2, b2, gate, ex_tile, live_tile):
    grid_spec = pltpu.PrefetchScalarGridSpec(
        num_scalar_prefetch=2,
        grid=(P_TILES, NH),
        in_specs=[
            pl.BlockSpec((T_ROW, D), lambda p, hb, ex, lv: (p, 0)),
            pl.BlockSpec((1, HBLK, D), lambda p, hb, ex, lv: (ex[p], hb, 0)),
            pl.BlockSpec((1, 1, 1, HBLK), lambda p, hb, ex, lv: (ex[p], hb, 0, 0)),
            pl.BlockSpec((1, D, HBLK), lambda p, hb, ex, lv: (ex[p], 0, hb)),
            pl.BlockSpec((1, 1, D), lambda p, hb, ex, lv: (ex[p], 0, 0)),
            pl.BlockSpec((T_ROW, 1), lambda p, hb, ex, lv: (p, 0)),
        ],
        out_specs=pl.BlockSpec((T_ROW, D), lambda p, hb, ex, lv: (p, 0)),
    )
    return pl.pallas_call(
        _ffn_body,
        grid_spec=grid_spec,
        out_shape=jax.ShapeDtypeStruct((CAP, D), jnp.float32),
        compiler_params=pltpu.CompilerParams(
            dimension_semantics=("arbitrary", "arbitrary")),
    )(ex_tile, live_tile, xs, w1, b1.reshape(E, NH, 1, HBLK), w2,
      b2.reshape(E, 1, D), gate)


def _shared_body(x_ref, w1_ref, b1_ref, w2_ref, b2_ref, hg_ref, out_ref):
    hb = pl.program_id(1)
    x = x_ref[...]
    h1 = lax.dot_general(x, w1_ref[0], (((1,), (1,)), ((), ())),
                         preferred_element_type=jnp.float32)
    h1 = jnp.maximum(h1 + b1_ref[...], 0.0)
    part = lax.dot_general(h1, w2_ref[0], (((1,), (1,)), ((), ())),
                           preferred_element_type=jnp.float32)

    @pl.when(hb == 0)
    def _():
        out_ref[...] = part

    @pl.when(hb > 0)
    def _():
        out_ref[...] += part

    @pl.when(hb == NH - 1)
    def _():
        out_ref[...] = (out_ref[...] + b2_ref[...]
                        + hg_ref[:, 0, :] + hg_ref[:, 1, :])


def _run_shared(xf, sw1, sb1, sw2, sb2, hg):
    nt = N_TOK // T_TOK
    return pl.pallas_call(
        _shared_body,
        grid=(nt, NH),
        in_specs=[
            pl.BlockSpec((T_TOK, D), lambda t, hb: (t, 0)),
            pl.BlockSpec((1, HBLK, D), lambda t, hb: (0, hb, 0)),
            pl.BlockSpec((1, HBLK), lambda t, hb: (0, hb)),
            pl.BlockSpec((1, D, HBLK), lambda t, hb: (0, 0, hb)),
            pl.BlockSpec((1, D), lambda t, hb: (0, 0)),
            pl.BlockSpec((T_TOK, 2, D), lambda t, hb: (t, 0, 0)),
        ],
        out_specs=pl.BlockSpec((T_TOK, D), lambda t, hb: (t, 0)),
        out_shape=jax.ShapeDtypeStruct((N_TOK, D), jnp.float32),
        compiler_params=pltpu.CompilerParams(
            dimension_semantics=("arbitrary", "arbitrary")),
    )(xf, sw1, sb1, sw2, sb2, hg)


def kernel(x, w_g, w_n, W1, b1, W2, b2, sW1, sb1, sW2, sb2):
    bsz, seq, d = x.shape
    xf = x.reshape(N_TOK, D)
    z = jax.random.normal(jax.random.key(42), (bsz, seq, E),
                          jnp.float32).reshape(N_TOK, E)

    i1, i2, g1, g2, score = _run_router(xf, w_g, w_n, z)

    eflat = jnp.concatenate([i1, i2], axis=1).reshape(-1)
    gflat = jnp.concatenate([g1, g2], axis=1).reshape(-1)
    onehot = (eflat[:, None] == jnp.arange(E)[None, :]).astype(jnp.int32)
    incl = jnp.cumsum(onehot, axis=0)
    rank = jnp.take_along_axis(incl, eflat[:, None], axis=1)[:, 0] - 1
    counts = incl[-1]
    aligned = ((counts + T_ROW - 1) // T_ROW) * T_ROW
    ends = jnp.cumsum(aligned)
    starts = ends - aligned
    slot = (starts[eflat] + rank).astype(jnp.int32)
    pair_tok = (jnp.arange(K * N_TOK, dtype=jnp.int32) // K)
    src_tok = jnp.zeros((CAP,), jnp.int32).at[slot].set(pair_tok)
    gate_slot = jnp.zeros((CAP, 1), jnp.float32).at[slot, 0].set(gflat)
    tile_start = jnp.arange(P_TILES, dtype=jnp.int32) * T_ROW
    ex_tile = jnp.minimum(
        jnp.sum((tile_start[:, None] >= ends[None, :]).astype(jnp.int32),
                axis=1), E - 1).astype(jnp.int32)
    live_tile = (tile_start < ends[-1]).astype(jnp.int32)

    xs = _gather_rows(xf, src_tok)
    h = _run_ffn(xs, W1, b1, W2, b2, gate_slot, ex_tile, live_tile)
    hg = _gather_rows(h, slot).reshape(N_TOK, K, D)
    out = _run_shared(xf, sW1, sb1, sW2, sb2, hg)

    return out.reshape(bsz, seq, d), score[0, 0]
